# baseline (speedup 1.0000x reference)
import jax
import jax.numpy as jnp
from jax.experimental import pallas as pl
from jax.experimental.pallas import tpu as pltpu

_NQ = 8
_K = 1024
_D = 256
_BLK = 2048


def _col_sumsq(x):
    s = x * x
    s = s[:128, :] + s[128:, :]
    acc = s[0:8, :]
    for j in range(1, 16):
        acc = acc + s[8 * j:8 * j + 8, :]
    acc = acc[:4, :] + acc[4:, :]
    acc = acc[:2, :] + acc[2:, :]
    return acc[:1, :] + acc[1:, :]


def _rvq_block_kernel(zt_ref, cb_ref, cbt_ref, qt_ref, idx_ref, loss_ref):
    rest = zt_ref[...]
    quant = jnp.zeros_like(rest)
    loss_sum = jnp.float32(0.0)
    sub_iota = jax.lax.broadcasted_iota(jnp.int32, (_K, _BLK), 0)

    cbn_rows = jnp.concatenate(
        [_col_sumsq(cbt_ref[i]) for i in range(_NQ)], axis=0)
    cbn_cols = cbn_rows.T

    for i in range(_NQ):
        rnorm = _col_sumsq(rest)
        prod2 = jax.lax.dot_general(
            cb_ref[i] * -2.0, rest, (((1,), (0,)), ((), ())),
            preferred_element_type=jnp.float32)
        d2 = (rnorm + prod2) + cbn_cols[:, i:i + 1]
        min_d = jnp.min(d2, axis=0, keepdims=True)
        idx = jnp.min(jnp.where(d2 == min_d, sub_iota, _K),
                      axis=0, keepdims=True)
        hi = jax.lax.shift_right_logical(idx, 7)
        lob = jnp.broadcast_to(jnp.bitwise_and(idx, 127),
                               (_D, _BLK))
        cbt_i = cbt_ref[i]
        ql = jnp.take_along_axis(cbt_i[:, 0:128], lob, axis=1)
        for g in range(1, 8):
            cand = jnp.take_along_axis(
                cbt_i[:, 128 * g:128 * (g + 1)], lob, axis=1)
            ql = jnp.where(hi == g, cand, ql)
        quant = quant + ql
        rest = rest - ql
        loss_sum = loss_sum + jnp.sum(rest * rest)
        idx_ref[i, :] = idx[0, :]
    qt_ref[...] = quant
    loss_ref[0, 0, 0] = loss_sum


def kernel(z, codebooks):
    B, T, D = z.shape
    ntok = B * T
    zt = z.reshape(ntok, D).T
    cbt = codebooks.transpose(0, 2, 1)
    nblocks = ntok // _BLK
    qt, idx, loss = pl.pallas_call(
        _rvq_block_kernel,
        grid=(nblocks,),
        in_specs=[
            pl.BlockSpec((_D, _BLK), lambda i: (0, i)),
            pl.BlockSpec((_NQ, _K, _D), lambda i: (0, 0, 0)),
            pl.BlockSpec((_NQ, _D, _K), lambda i: (0, 0, 0)),
        ],
        out_specs=[
            pl.BlockSpec((_D, _BLK), lambda i: (0, i)),
            pl.BlockSpec((_NQ, _BLK), lambda i: (0, i)),
            pl.BlockSpec((1, 1, 1), lambda i: (i, 0, 0),
                         memory_space=pltpu.SMEM),
        ],
        out_shape=[
            jax.ShapeDtypeStruct((_D, ntok), jnp.float32),
            jax.ShapeDtypeStruct((_NQ, ntok), jnp.int32),
            jax.ShapeDtypeStruct((nblocks, 1, 1), jnp.float32),
        ],
        compiler_params=pltpu.CompilerParams(
            dimension_semantics=("parallel",)),
    )(zt, codebooks, cbt)
    quantized_st = qt.T.reshape(B, T, D)
    indices = idx.reshape(_NQ, B, T).transpose(1, 0, 2)
    commitment_loss = jnp.sum(loss) / jnp.float32(_NQ * ntok * _D)
    return quantized_st, indices, commitment_loss


# speedup vs baseline: 1.7931x; 1.7931x over previous
import jax
import jax.numpy as jnp
from jax.experimental import pallas as pl
from jax.experimental.pallas import tpu as pltpu

_NQ = 8
_K = 1024
_D = 256
_BLK = 2048


def _col_sumsq(x):
    s = x * x
    s = s[:128, :] + s[128:, :]
    acc = s[0:8, :]
    for j in range(1, 16):
        acc = acc + s[8 * j:8 * j + 8, :]
    acc = acc[:4, :] + acc[4:, :]
    acc = acc[:2, :] + acc[2:, :]
    return acc[:1, :] + acc[1:, :]


def _rvq_block_kernel(zt_ref, cb_ref, cbt_ref, qt_ref, idx_ref, loss_ref):
    rest = zt_ref[...]
    quant = jnp.zeros_like(rest)
    loss_sum = jnp.float32(0.0)
    sub_iota = jax.lax.broadcasted_iota(jnp.int32, (_K, _BLK), 0)

    cbn_rows = jnp.concatenate(
        [_col_sumsq(cbt_ref[i]) for i in range(_NQ)], axis=0)
    cbn_cols = cbn_rows.T

    for i in range(_NQ):
        rnorm = _col_sumsq(rest)
        prod2 = jax.lax.dot_general(
            cb_ref[i] * -2.0, rest, (((1,), (0,)), ((), ())),
            preferred_element_type=jnp.float32)
        d2 = (rnorm + prod2) + cbn_cols[:, i:i + 1]
        min_d = jnp.min(d2, axis=0, keepdims=True)
        idx = jnp.min(jnp.where(d2 == min_d, sub_iota, _K),
                      axis=0, keepdims=True)
        onehot = (sub_iota == idx).astype(jnp.bfloat16)
        cbt_i = cbt_ref[i]
        cb_h = cbt_i.astype(jnp.bfloat16)
        r1 = cbt_i - cb_h.astype(jnp.float32)
        cb_m = r1.astype(jnp.bfloat16)
        cb_l = (r1 - cb_m.astype(jnp.float32)).astype(jnp.bfloat16)
        dims = (((1,), (0,)), ((), ()))
        mm_h = jax.lax.dot_general(cb_h, onehot, dims,
                                   preferred_element_type=jnp.float32)
        mm_m = jax.lax.dot_general(cb_m, onehot, dims,
                                   preferred_element_type=jnp.float32)
        mm_l = jax.lax.dot_general(cb_l, onehot, dims,
                                   preferred_element_type=jnp.float32)
        ql = (mm_h + mm_m) + mm_l
        quant = quant + ql
        rest = rest - ql
        loss_sum = loss_sum + jnp.sum(rest * rest)
        idx_ref[i, :] = idx[0, :]
    qt_ref[...] = quant
    loss_ref[0, 0, 0] = loss_sum


def kernel(z, codebooks):
    B, T, D = z.shape
    ntok = B * T
    zt = z.reshape(ntok, D).T
    cbt = codebooks.transpose(0, 2, 1)
    nblocks = ntok // _BLK
    qt, idx, loss = pl.pallas_call(
        _rvq_block_kernel,
        grid=(nblocks,),
        in_specs=[
            pl.BlockSpec((_D, _BLK), lambda i: (0, i)),
            pl.BlockSpec((_NQ, _K, _D), lambda i: (0, 0, 0)),
            pl.BlockSpec((_NQ, _D, _K), lambda i: (0, 0, 0)),
        ],
        out_specs=[
            pl.BlockSpec((_D, _BLK), lambda i: (0, i)),
            pl.BlockSpec((_NQ, _BLK), lambda i: (0, i)),
            pl.BlockSpec((1, 1, 1), lambda i: (i, 0, 0),
                         memory_space=pltpu.SMEM),
        ],
        out_shape=[
            jax.ShapeDtypeStruct((_D, ntok), jnp.float32),
            jax.ShapeDtypeStruct((_NQ, ntok), jnp.int32),
            jax.ShapeDtypeStruct((nblocks, 1, 1), jnp.float32),
        ],
        compiler_params=pltpu.CompilerParams(
            dimension_semantics=("parallel",)),
    )(zt, codebooks, cbt)
    quantized_st = qt.T.reshape(B, T, D)
    indices = idx.reshape(_NQ, B, T).transpose(1, 0, 2)
    commitment_loss = jnp.sum(loss) / jnp.float32(_NQ * ntok * _D)
    return quantized_st, indices, commitment_loss


# speedup vs baseline: 1.8106x; 1.0098x over previous
import jax
import jax.numpy as jnp
from jax.experimental import pallas as pl
from jax.experimental.pallas import tpu as pltpu

_NQ = 8
_K = 1024
_D = 256
_BLK = 2048
_H = _BLK // 2


def _col_sumsq(x):
    s = x * x
    s = s[:128, :] + s[128:, :]
    acc = s[0:8, :]
    for j in range(1, 16):
        acc = acc + s[8 * j:8 * j + 8, :]
    acc = acc[:4, :] + acc[4:, :]
    acc = acc[:2, :] + acc[2:, :]
    return acc[:1, :] + acc[1:, :]


def _stage(rest, cb2, cb_h, cb_m, cb_l, cbn_col, sub_iota):
    rnorm = _col_sumsq(rest)
    prod2 = jax.lax.dot_general(
        cb2, rest, (((1,), (0,)), ((), ())),
        preferred_element_type=jnp.float32)
    d2 = (rnorm + prod2) + cbn_col
    min_d = jnp.min(d2, axis=0, keepdims=True)
    idx = jnp.min(jnp.where(d2 == min_d, sub_iota, _K),
                  axis=0, keepdims=True)
    onehot = (sub_iota == idx).astype(jnp.bfloat16)
    dims = (((1,), (0,)), ((), ()))
    mm_h = jax.lax.dot_general(cb_h, onehot, dims,
                               preferred_element_type=jnp.float32)
    mm_m = jax.lax.dot_general(cb_m, onehot, dims,
                               preferred_element_type=jnp.float32)
    mm_l = jax.lax.dot_general(cb_l, onehot, dims,
                               preferred_element_type=jnp.float32)
    ql = (mm_h + mm_m) + mm_l
    return idx, ql


def _rvq_block_kernel(zt_ref, cb_ref, cbt_ref, qt_ref, idx_ref, loss_ref):
    rest_a = zt_ref[:, :_H]
    rest_b = zt_ref[:, _H:]
    quant_a = jnp.zeros_like(rest_a)
    quant_b = jnp.zeros_like(rest_b)
    loss_sum = jnp.float32(0.0)
    sub_iota = jax.lax.broadcasted_iota(jnp.int32, (_K, _H), 0)

    cbn_rows = jnp.concatenate(
        [_col_sumsq(cbt_ref[i]) for i in range(_NQ)], axis=0)
    cbn_cols = cbn_rows.T

    for i in range(_NQ):
        cb2 = cb_ref[i] * -2.0
        cbt_i = cbt_ref[i]
        cb_h = cbt_i.astype(jnp.bfloat16)
        r1 = cbt_i - cb_h.astype(jnp.float32)
        cb_m = r1.astype(jnp.bfloat16)
        cb_l = (r1 - cb_m.astype(jnp.float32)).astype(jnp.bfloat16)
        cbn_col = cbn_cols[:, i:i + 1]
        idx_a, ql_a = _stage(rest_a, cb2, cb_h, cb_m, cb_l, cbn_col, sub_iota)
        idx_b, ql_b = _stage(rest_b, cb2, cb_h, cb_m, cb_l, cbn_col, sub_iota)
        quant_a = quant_a + ql_a
        rest_a = rest_a - ql_a
        quant_b = quant_b + ql_b
        rest_b = rest_b - ql_b
        loss_sum = (loss_sum + jnp.sum(rest_a * rest_a)
                    + jnp.sum(rest_b * rest_b))
        idx_ref[i, :_H] = idx_a[0, :]
        idx_ref[i, _H:] = idx_b[0, :]
    qt_ref[:, :_H] = quant_a
    qt_ref[:, _H:] = quant_b
    loss_ref[0, 0, 0] = loss_sum


def kernel(z, codebooks):
    B, T, D = z.shape
    ntok = B * T
    zt = z.reshape(ntok, D).T
    cbt = codebooks.transpose(0, 2, 1)
    nblocks = ntok // _BLK
    qt, idx, loss = pl.pallas_call(
        _rvq_block_kernel,
        grid=(nblocks,),
        in_specs=[
            pl.BlockSpec((_D, _BLK), lambda i: (0, i)),
            pl.BlockSpec((_NQ, _K, _D), lambda i: (0, 0, 0)),
            pl.BlockSpec((_NQ, _D, _K), lambda i: (0, 0, 0)),
        ],
        out_specs=[
            pl.BlockSpec((_D, _BLK), lambda i: (0, i)),
            pl.BlockSpec((_NQ, _BLK), lambda i: (0, i)),
            pl.BlockSpec((1, 1, 1), lambda i: (i, 0, 0),
                         memory_space=pltpu.SMEM),
        ],
        out_shape=[
            jax.ShapeDtypeStruct((_D, ntok), jnp.float32),
            jax.ShapeDtypeStruct((_NQ, ntok), jnp.int32),
            jax.ShapeDtypeStruct((nblocks, 1, 1), jnp.float32),
        ],
        compiler_params=pltpu.CompilerParams(
            dimension_semantics=("parallel",)),
    )(zt, codebooks, cbt)
    quantized_st = qt.T.reshape(B, T, D)
    indices = idx.reshape(_NQ, B, T).transpose(1, 0, 2)
    commitment_loss = jnp.sum(loss) / jnp.float32(_NQ * ntok * _D)
    return quantized_st, indices, commitment_loss


# speedup vs baseline: 1.8338x; 1.0128x over previous
import jax
import jax.numpy as jnp
from jax.experimental import pallas as pl
from jax.experimental.pallas import tpu as pltpu

_NQ = 8
_K = 1024
_D = 256
_BLK = 2048
_H = _BLK // 2


def _col_sumsq(x):
    s = x * x
    s = s[:128, :] + s[128:, :]
    acc = s[0:8, :]
    for j in range(1, 16):
        acc = acc + s[8 * j:8 * j + 8, :]
    acc = acc[:4, :] + acc[4:, :]
    acc = acc[:2, :] + acc[2:, :]
    return acc[:1, :] + acc[1:, :]


def _prelude_kernel(cb_ref, cbt_ref, cb2_ref, cbh_ref, cbm_ref, cbl_ref,
                    cbn_ref):
    for i in range(_NQ):
        cb2_ref[i] = cb_ref[i] * -2.0
        cbt_i = cbt_ref[i]
        cb_h = cbt_i.astype(jnp.bfloat16)
        r1 = cbt_i - cb_h.astype(jnp.float32)
        cb_m = r1.astype(jnp.bfloat16)
        cbh_ref[i] = cb_h
        cbm_ref[i] = cb_m
        cbl_ref[i] = (r1 - cb_m.astype(jnp.float32)).astype(jnp.bfloat16)
    cbn_rows = jnp.concatenate(
        [_col_sumsq(cbt_ref[i]) for i in range(_NQ)], axis=0)
    cbn_ref[...] = cbn_rows.T


def _stage(rest, cb2, cb_h, cb_m, cb_l, cbn_col, sub_iota):
    rnorm = _col_sumsq(rest)
    prod2 = jax.lax.dot_general(
        cb2, rest, (((1,), (0,)), ((), ())),
        preferred_element_type=jnp.float32)
    d2 = (rnorm + prod2) + cbn_col
    min_d = jnp.min(d2, axis=0, keepdims=True)
    idx = jnp.min(jnp.where(d2 == min_d, sub_iota, _K),
                  axis=0, keepdims=True)
    onehot = (sub_iota == idx).astype(jnp.bfloat16)
    dims = (((1,), (0,)), ((), ()))
    mm_h = jax.lax.dot_general(cb_h, onehot, dims,
                               preferred_element_type=jnp.float32)
    mm_m = jax.lax.dot_general(cb_m, onehot, dims,
                               preferred_element_type=jnp.float32)
    mm_l = jax.lax.dot_general(cb_l, onehot, dims,
                               preferred_element_type=jnp.float32)
    ql = (mm_h + mm_m) + mm_l
    return idx, ql


def _rvq_block_kernel(zt_ref, cb2_ref, cbh_ref, cbm_ref, cbl_ref, cbn_ref,
                      qt_ref, idx_ref, loss_ref):
    rest_a = zt_ref[:, :_H]
    rest_b = zt_ref[:, _H:]
    quant_a = jnp.zeros_like(rest_a)
    quant_b = jnp.zeros_like(rest_b)
    loss_sum = jnp.float32(0.0)
    sub_iota = jax.lax.broadcasted_iota(jnp.int32, (_K, _H), 0)

    for i in range(_NQ):
        cb2 = cb2_ref[i]
        cb_h = cbh_ref[i]
        cb_m = cbm_ref[i]
        cb_l = cbl_ref[i]
        cbn_col = cbn_ref[:, i:i + 1]
        idx_a, ql_a = _stage(rest_a, cb2, cb_h, cb_m, cb_l, cbn_col, sub_iota)
        idx_b, ql_b = _stage(rest_b, cb2, cb_h, cb_m, cb_l, cbn_col, sub_iota)
        quant_a = quant_a + ql_a
        rest_a = rest_a - ql_a
        quant_b = quant_b + ql_b
        rest_b = rest_b - ql_b
        loss_sum = (loss_sum + jnp.sum(rest_a * rest_a)
                    + jnp.sum(rest_b * rest_b))
        idx_ref[i, :_H] = idx_a[0, :]
        idx_ref[i, _H:] = idx_b[0, :]
    qt_ref[:, :_H] = quant_a
    qt_ref[:, _H:] = quant_b
    loss_ref[0, 0, 0] = loss_sum


def kernel(z, codebooks):
    B, T, D = z.shape
    ntok = B * T
    zt = z.reshape(ntok, D).T
    cbt = codebooks.transpose(0, 2, 1)
    nblocks = ntok // _BLK

    cb2, cbh, cbm, cbl, cbn = pl.pallas_call(
        _prelude_kernel,
        out_shape=[
            jax.ShapeDtypeStruct((_NQ, _K, _D), jnp.float32),
            jax.ShapeDtypeStruct((_NQ, _D, _K), jnp.bfloat16),
            jax.ShapeDtypeStruct((_NQ, _D, _K), jnp.bfloat16),
            jax.ShapeDtypeStruct((_NQ, _D, _K), jnp.bfloat16),
            jax.ShapeDtypeStruct((_K, _NQ), jnp.float32),
        ],
    )(codebooks, cbt)

    qt, idx, loss = pl.pallas_call(
        _rvq_block_kernel,
        grid=(nblocks,),
        in_specs=[
            pl.BlockSpec((_D, _BLK), lambda i: (0, i)),
            pl.BlockSpec((_NQ, _K, _D), lambda i: (0, 0, 0)),
            pl.BlockSpec((_NQ, _D, _K), lambda i: (0, 0, 0)),
            pl.BlockSpec((_NQ, _D, _K), lambda i: (0, 0, 0)),
            pl.BlockSpec((_NQ, _D, _K), lambda i: (0, 0, 0)),
            pl.BlockSpec((_K, _NQ), lambda i: (0, 0)),
        ],
        out_specs=[
            pl.BlockSpec((_D, _BLK), lambda i: (0, i)),
            pl.BlockSpec((_NQ, _BLK), lambda i: (0, i)),
            pl.BlockSpec((1, 1, 1), lambda i: (i, 0, 0),
                         memory_space=pltpu.SMEM),
        ],
        out_shape=[
            jax.ShapeDtypeStruct((_D, ntok), jnp.float32),
            jax.ShapeDtypeStruct((_NQ, ntok), jnp.int32),
            jax.ShapeDtypeStruct((nblocks, 1, 1), jnp.float32),
        ],
        compiler_params=pltpu.CompilerParams(
            dimension_semantics=("parallel",)),
    )(zt, cb2, cbh, cbm, cbl, cbn)
    quantized_st = qt.T.reshape(B, T, D)
    indices = idx.reshape(_NQ, B, T).transpose(1, 0, 2)
    commitment_loss = jnp.sum(loss) / jnp.float32(_NQ * ntok * _D)
    return quantized_st, indices, commitment_loss


# speedup vs baseline: 2.0837x; 1.1363x over previous
import jax
import jax.numpy as jnp
from jax.experimental import pallas as pl
from jax.experimental.pallas import tpu as pltpu

_NQ = 8
_K = 1024
_D = 256
_BLK = 2048
_H = _BLK // 2


def _col_sumsq(x):
    s = x * x
    s = s[:128, :] + s[128:, :]
    acc = s[0:8, :]
    for j in range(1, 16):
        acc = acc + s[8 * j:8 * j + 8, :]
    acc = acc[:4, :] + acc[4:, :]
    acc = acc[:2, :] + acc[2:, :]
    return acc[:1, :] + acc[1:, :]


def _prelude_kernel(cb_ref, cbt_ref, cb2_ref, cbh_ref, cbm_ref, cbn_ref):
    for i in range(_NQ):
        cb2_ref[i] = cb_ref[i] * -2.0
        cbt_i = cbt_ref[i]
        cb_h = cbt_i.astype(jnp.bfloat16)
        r1 = cbt_i - cb_h.astype(jnp.float32)
        cbh_ref[i] = cb_h
        cbm_ref[i] = r1.astype(jnp.bfloat16)
    cbn_rows = jnp.concatenate(
        [_col_sumsq(cbt_ref[i]) for i in range(_NQ)], axis=0)
    cbn_ref[...] = cbn_rows.T


def _stage(rest, cb2, cb_h, cb_m, cbn_col, sub_iota):
    rnorm = _col_sumsq(rest)
    prod2 = jax.lax.dot_general(
        cb2, rest, (((1,), (0,)), ((), ())),
        preferred_element_type=jnp.float32)
    d2 = (rnorm + prod2) + cbn_col
    min_d = jnp.min(d2, axis=0, keepdims=True)
    idx = jnp.min(jnp.where(d2 == min_d, sub_iota, _K),
                  axis=0, keepdims=True)
    onehot = (sub_iota == idx).astype(jnp.bfloat16)
    dims = (((1,), (0,)), ((), ()))
    mm_h = jax.lax.dot_general(cb_h, onehot, dims,
                               preferred_element_type=jnp.float32)
    mm_m = jax.lax.dot_general(cb_m, onehot, dims,
                               preferred_element_type=jnp.float32)
    ql = mm_h + mm_m
    return idx, ql


def _rvq_block_kernel(zt_ref, cb2_ref, cbh_ref, cbm_ref, cbn_ref,
                      qt_ref, idx_ref, loss_ref):
    rest_a = zt_ref[:, :_H]
    rest_b = zt_ref[:, _H:]
    quant_a = jnp.zeros_like(rest_a)
    quant_b = jnp.zeros_like(rest_b)
    loss_sum = jnp.float32(0.0)
    sub_iota = jax.lax.broadcasted_iota(jnp.int32, (_K, _H), 0)

    for i in range(_NQ):
        cb2 = cb2_ref[i]
        cb_h = cbh_ref[i]
        cb_m = cbm_ref[i]
        cbn_col = cbn_ref[:, i:i + 1]
        idx_a, ql_a = _stage(rest_a, cb2, cb_h, cb_m, cbn_col, sub_iota)
        idx_b, ql_b = _stage(rest_b, cb2, cb_h, cb_m, cbn_col, sub_iota)
        quant_a = quant_a + ql_a
        rest_a = rest_a - ql_a
        quant_b = quant_b + ql_b
        rest_b = rest_b - ql_b
        loss_sum = (loss_sum + jnp.sum(rest_a * rest_a)
                    + jnp.sum(rest_b * rest_b))
        idx_ref[i, :_H] = idx_a[0, :]
        idx_ref[i, _H:] = idx_b[0, :]
    qt_ref[:, :_H] = quant_a
    qt_ref[:, _H:] = quant_b
    loss_ref[0, 0, 0] = loss_sum


def kernel(z, codebooks):
    B, T, D = z.shape
    ntok = B * T
    zt = z.reshape(ntok, D).T
    cbt = codebooks.transpose(0, 2, 1)
    nblocks = ntok // _BLK

    cb2, cbh, cbm, cbn = pl.pallas_call(
        _prelude_kernel,
        out_shape=[
            jax.ShapeDtypeStruct((_NQ, _K, _D), jnp.float32),
            jax.ShapeDtypeStruct((_NQ, _D, _K), jnp.bfloat16),
            jax.ShapeDtypeStruct((_NQ, _D, _K), jnp.bfloat16),
            jax.ShapeDtypeStruct((_K, _NQ), jnp.float32),
        ],
    )(codebooks, cbt)

    qt, idx, loss = pl.pallas_call(
        _rvq_block_kernel,
        grid=(nblocks,),
        in_specs=[
            pl.BlockSpec((_D, _BLK), lambda i: (0, i)),
            pl.BlockSpec((_NQ, _K, _D), lambda i: (0, 0, 0)),
            pl.BlockSpec((_NQ, _D, _K), lambda i: (0, 0, 0)),
            pl.BlockSpec((_NQ, _D, _K), lambda i: (0, 0, 0)),
            pl.BlockSpec((_K, _NQ), lambda i: (0, 0)),
        ],
        out_specs=[
            pl.BlockSpec((_D, _BLK), lambda i: (0, i)),
            pl.BlockSpec((_NQ, _BLK), lambda i: (0, i)),
            pl.BlockSpec((1, 1, 1), lambda i: (i, 0, 0),
                         memory_space=pltpu.SMEM),
        ],
        out_shape=[
            jax.ShapeDtypeStruct((_D, ntok), jnp.float32),
            jax.ShapeDtypeStruct((_NQ, ntok), jnp.int32),
            jax.ShapeDtypeStruct((nblocks, 1, 1), jnp.float32),
        ],
        compiler_params=pltpu.CompilerParams(
            dimension_semantics=("parallel",)),
    )(zt, cb2, cbh, cbm, cbn)
    quantized_st = qt.T.reshape(B, T, D)
    indices = idx.reshape(_NQ, B, T).transpose(1, 0, 2)
    commitment_loss = jnp.sum(loss) / jnp.float32(_NQ * ntok * _D)
    return quantized_st, indices, commitment_loss


# speedup vs baseline: 2.1338x; 1.0240x over previous
import jax
import jax.numpy as jnp
from jax.experimental import pallas as pl
from jax.experimental.pallas import tpu as pltpu

_NQ = 8
_K = 1024
_D = 256
_BLK = 2048
_H = _BLK // 2


def _col_sumsq(x):
    s = x * x
    s = s[:128, :] + s[128:, :]
    acc = s[0:8, :]
    for j in range(1, 16):
        acc = acc + s[8 * j:8 * j + 8, :]
    acc = acc[:4, :] + acc[4:, :]
    acc = acc[:2, :] + acc[2:, :]
    return acc[:1, :] + acc[1:, :]


def _prelude_kernel(cb_ref, cbt_ref, cb2_ref, cbh_ref, cbm_ref, cbn_ref):
    for i in range(_NQ):
        cb2_ref[i] = cb_ref[i] * -2.0
        cbt_i = cbt_ref[i]
        cb_h = cbt_i.astype(jnp.bfloat16)
        r1 = cbt_i - cb_h.astype(jnp.float32)
        cbh_ref[i] = cb_h
        cbm_ref[i] = r1.astype(jnp.bfloat16)
    cbn_rows = jnp.concatenate(
        [_col_sumsq(cbt_ref[i]) for i in range(_NQ)], axis=0)
    cbn_ref[...] = cbn_rows.T


def _stage(rest, cb2, cb_h, cb_m, cbn_col, sub_iota):
    rnorm = _col_sumsq(rest)
    prod2 = jax.lax.dot_general(
        cb2, rest, (((1,), (0,)), ((), ())),
        preferred_element_type=jnp.float32)
    d2 = (rnorm + prod2) + cbn_col
    min_d = jnp.min(d2, axis=0, keepdims=True)
    idx = jnp.min(jnp.where(d2 == min_d, sub_iota, _K),
                  axis=0, keepdims=True)
    onehot = (sub_iota == idx).astype(jnp.bfloat16)
    dims = (((1,), (0,)), ((), ()))
    mm_h = jax.lax.dot_general(cb_h, onehot, dims,
                               preferred_element_type=jnp.float32)
    mm_m = jax.lax.dot_general(cb_m, onehot, dims,
                               preferred_element_type=jnp.float32)
    ql = mm_h + mm_m
    return idx, ql, rnorm


def _rvq_block_kernel(zt_ref, cb2_ref, cbh_ref, cbm_ref, cbn_ref,
                      qt_ref, idx_ref, loss_ref):
    rest_a = zt_ref[:, :_H]
    rest_b = zt_ref[:, _H:]
    quant_a = jnp.zeros_like(rest_a)
    quant_b = jnp.zeros_like(rest_b)
    loss_sum = jnp.float32(0.0)
    sub_iota = jax.lax.broadcasted_iota(jnp.int32, (_K, 1), 0)

    for i in range(_NQ):
        cb2 = cb2_ref[i]
        cb_h = cbh_ref[i]
        cb_m = cbm_ref[i]
        cbn_col = cbn_ref[:, i:i + 1]
        idx_a, ql_a, rn_a = _stage(rest_a, cb2, cb_h, cb_m, cbn_col, sub_iota)
        idx_b, ql_b, rn_b = _stage(rest_b, cb2, cb_h, cb_m, cbn_col, sub_iota)
        if i > 0:
            loss_sum = loss_sum + jnp.sum(rn_a) + jnp.sum(rn_b)
        quant_a = quant_a + ql_a
        rest_a = rest_a - ql_a
        quant_b = quant_b + ql_b
        rest_b = rest_b - ql_b
        idx_ref[i, :_H] = idx_a[0, :]
        idx_ref[i, _H:] = idx_b[0, :]
    loss_sum = (loss_sum + jnp.sum(_col_sumsq(rest_a))
                + jnp.sum(_col_sumsq(rest_b)))
    qt_ref[:, :_H] = quant_a
    qt_ref[:, _H:] = quant_b
    loss_ref[0, 0, 0] = loss_sum


def kernel(z, codebooks):
    B, T, D = z.shape
    ntok = B * T
    zt = z.reshape(ntok, D).T
    cbt = codebooks.transpose(0, 2, 1)
    nblocks = ntok // _BLK

    cb2, cbh, cbm, cbn = pl.pallas_call(
        _prelude_kernel,
        out_shape=[
            jax.ShapeDtypeStruct((_NQ, _K, _D), jnp.float32),
            jax.ShapeDtypeStruct((_NQ, _D, _K), jnp.bfloat16),
            jax.ShapeDtypeStruct((_NQ, _D, _K), jnp.bfloat16),
            jax.ShapeDtypeStruct((_K, _NQ), jnp.float32),
        ],
    )(codebooks, cbt)

    qt, idx, loss = pl.pallas_call(
        _rvq_block_kernel,
        grid=(nblocks,),
        in_specs=[
            pl.BlockSpec((_D, _BLK), lambda i: (0, i)),
            pl.BlockSpec((_NQ, _K, _D), lambda i: (0, 0, 0)),
            pl.BlockSpec((_NQ, _D, _K), lambda i: (0, 0, 0)),
            pl.BlockSpec((_NQ, _D, _K), lambda i: (0, 0, 0)),
            pl.BlockSpec((_K, _NQ), lambda i: (0, 0)),
        ],
        out_specs=[
            pl.BlockSpec((_D, _BLK), lambda i: (0, i)),
            pl.BlockSpec((_NQ, _BLK), lambda i: (0, i)),
            pl.BlockSpec((1, 1, 1), lambda i: (i, 0, 0),
                         memory_space=pltpu.SMEM),
        ],
        out_shape=[
            jax.ShapeDtypeStruct((_D, ntok), jnp.float32),
            jax.ShapeDtypeStruct((_NQ, ntok), jnp.int32),
            jax.ShapeDtypeStruct((nblocks, 1, 1), jnp.float32),
        ],
        compiler_params=pltpu.CompilerParams(
            dimension_semantics=("parallel",)),
    )(zt, cb2, cbh, cbm, cbn)
    quantized_st = qt.T.reshape(B, T, D)
    indices = idx.reshape(_NQ, B, T).transpose(1, 0, 2)
    commitment_loss = jnp.sum(loss) / jnp.float32(_NQ * ntok * _D)
    return quantized_st, indices, commitment_loss
